# baseline (speedup 1.0000x reference)
import jax
import jax.numpy as jnp
from jax import lax
from jax.experimental import pallas as pl
from jax.experimental.pallas import tpu as pltpu
from jax.experimental.pallas import tpu_sc as plsc

_B = 4096
_F = 32
_PACK = 4
_NC, _NS, _NL = 2, 16, 16
_NW = _NC * _NS
_BPW = _B // _NW


def _mf_body(users_hbm, items_hbm, uf_hbm, if_hbm, ub_hbm, ib_hbm, out_hbm,
             uidx_v, iidx_v, ublk_v, iblk_v, ufb_v, ifb_v, ubr_v, ibr_v,
             out_v, sem):
    wid = lax.axis_index("s") * _NC + lax.axis_index("c")
    base = wid * _BPW

    pltpu.sync_copy(users_hbm.at[pl.ds(base, _BPW)], uidx_v)
    pltpu.sync_copy(items_hbm.at[pl.ds(base, _BPW)], iidx_v)

    for g in range(_BPW // _NL):
        s = pl.ds(g * _NL, _NL)
        ublk_v[s] = lax.shift_right_logical(uidx_v[s], 2)
        iblk_v[s] = lax.shift_right_logical(iidx_v[s], 2)

    cps = [
        pltpu.async_copy(uf_hbm.at[ublk_v], ufb_v, sem),
        pltpu.async_copy(if_hbm.at[iblk_v], ifb_v, sem),
        pltpu.async_copy(ub_hbm.at[uidx_v], ubr_v, sem),
        pltpu.async_copy(ib_hbm.at[iidx_v], ibr_v, sem),
    ]
    for cp in cps:
        cp.wait()

    lane = lax.iota(jnp.int32, _NL)
    for g in range(_BPW // _NL):
        s = pl.ds(g * _NL, _NL)
        row = lane + (g * _NL)
        ucol0 = lax.shift_left(lax.bitwise_and(uidx_v[s], _PACK - 1), 5)
        icol0 = lax.shift_left(lax.bitwise_and(iidx_v[s], _PACK - 1), 5)
        acc = ubr_v[s] + ibr_v[s] + 3.5
        for f in range(_F):
            stag = lax.bitwise_and(lane + f, _F - 1)
            u = plsc.load_gather(ufb_v, [row, ucol0 + stag])
            v = plsc.load_gather(ifb_v, [row, icol0 + stag])
            acc = acc + u * v
        out_v[s] = acc

    pltpu.sync_copy(out_v, out_hbm.at[pl.ds(base, _BPW)])


@jax.jit
def _mf(users, items, user_factors, item_factors, user_biases, item_biases):
    run = pl.kernel(
        _mf_body,
        out_type=jax.ShapeDtypeStruct((_B,), jnp.float32),
        mesh=plsc.VectorSubcoreMesh(core_axis_name="c", subcore_axis_name="s"),
        compiler_params=pltpu.CompilerParams(needs_layout_passes=False),
        scratch_types=[
            pltpu.VMEM((_BPW,), jnp.int32),
            pltpu.VMEM((_BPW,), jnp.int32),
            pltpu.VMEM((_BPW,), jnp.int32),
            pltpu.VMEM((_BPW,), jnp.int32),
            pltpu.VMEM((_BPW, _F * _PACK), jnp.float32),
            pltpu.VMEM((_BPW, _F * _PACK), jnp.float32),
            pltpu.VMEM((_BPW,), jnp.float32),
            pltpu.VMEM((_BPW,), jnp.float32),
            pltpu.VMEM((_BPW,), jnp.float32),
            pltpu.SemaphoreType.DMA,
        ],
    )
    return run(users, items,
               user_factors.reshape(-1, _F * _PACK),
               item_factors.reshape(-1, _F * _PACK),
               user_biases.reshape(-1), item_biases.reshape(-1))


def kernel(users, items, user_factors, item_factors, user_biases, item_biases):
    return _mf(users, items, user_factors, item_factors, user_biases,
               item_biases)
